# Initial kernel scaffold; baseline (speedup 1.0000x reference)
import jax
import jax.numpy as jnp
from jax.experimental import pallas as pl


def kernel(x, x_net, net_inst_adj, inst_net_adj_v_drive, inst_net_adj_v_sink, batch, num_vn, params):
    raise NotImplementedError("write your pallas kernel here")


# speedup vs baseline: 1.0926x; 1.0926x over previous
import functools

import jax
import jax.numpy as jnp
from jax.experimental import pallas as pl

N_INST = 8192
N_NET = 4096
EMB = 64
NUM_VN = 16

BR = 256
BN = 256


def _lrelu(v):
    return jnp.where(v >= 0, v, 0.1 * v)


def _dot(a, b):
    return jnp.dot(a, b, preferred_element_type=jnp.float32)


def _enc_inst_body(x_ref, w1_ref, b1_ref, w2_ref, b2_ref, o_ref):
    h = _lrelu(_dot(x_ref[...], w1_ref[...].T) + b1_ref[...])
    o_ref[...] = _lrelu(_dot(h, w2_ref[...].T) + b2_ref[...])


def _enc_net_body(x_ref, w1_ref, b1_ref, w2_ref, b2_ref,
                  p1_ref, pb1_ref, p2_ref, pb2_ref, hn_ref, na_ref):
    h = _lrelu(_dot(x_ref[...], w1_ref[...].T) + b1_ref[...])
    hn = _lrelu(_dot(h, w2_ref[...].T) + b2_ref[...])
    hn_ref[...] = hn
    t = jax.nn.relu(_dot(hn, p1_ref[...].T) + pb1_ref[...])
    na_ref[...] = _dot(t, p2_ref[...].T) + pb2_ref[...]


def _layer_body(drive_ref, sink_ref, na_ref, hb_ref, oh_ref, vnt_ref,
                psi1_ref, psib1_ref, psi2_ref, psib2_ref,
                m1_ref, mb1_ref, m2_ref, mb2_ref, g_ref, be_ref,
                q1_ref, qb1_ref, q2_ref, qb2_ref,
                ho_ref, hin_ref, hpre_ref, pool_ref, vn_next_ref, *, do_vn):
    i = pl.program_id(0)
    h_in = hb_ref[...] + _dot(oh_ref[...], vnt_ref[...])
    hin_ref[...] = h_in
    hd = _dot(drive_ref[...], na_ref[...])
    hs0 = _dot(sink_ref[...], na_ref[...])
    hs = _dot(jax.nn.relu(_dot(hs0, psi1_ref[...].T) + psib1_ref[...]),
              psi2_ref[...].T) + psib2_ref[...]
    hc = jnp.concatenate([h_in, hd, hs], axis=1)
    hm = jax.nn.relu(_dot(hc, m1_ref[...].T) + mb1_ref[...])
    ho = _dot(hm, m2_ref[...].T) + mb2_ref[...]
    hpre_ref[...] = ho
    mu = jnp.mean(ho, axis=-1, keepdims=True)
    var = jnp.mean((ho - mu) ** 2, axis=-1, keepdims=True)
    ho = (ho - mu) / jnp.sqrt(var + 1e-5) * g_ref[...] + be_ref[...]
    ho_ref[...] = _lrelu(ho) + h_in

    if do_vn:
        ones = jnp.ones((h_in.shape[0], EMB), jnp.float32)
        hp = jnp.concatenate([h_in, ones], axis=1)
        contrib = _dot(oh_ref[...].T, hp)

        @pl.when(i == 0)
        def _():
            pool_ref[...] = jnp.zeros_like(pool_ref)

        pool_ref[...] += contrib

        @pl.when(i == pl.num_programs(0) - 1)
        def _():
            pool = pool_ref[...]
            counts = jnp.maximum(pool[:, EMB:EMB + 1], 1.0)
            vn_in = pool[:, :EMB] / counts + vnt_ref[...]
            t = _lrelu(_dot(vn_in, q1_ref[...].T) + qb1_ref[...])
            vn_next_ref[...] = _lrelu(_dot(t, q2_ref[...].T) + qb2_ref[...])
    else:
        @pl.when(i == 0)
        def _():
            pool_ref[...] = jnp.zeros_like(pool_ref)
            vn_next_ref[...] = jnp.zeros_like(vn_next_ref)


def _net_body(adj_ref, ho_ref, hn_ref, p1_ref, pb1_ref, p2_ref, pb2_ref,
              hn1_ref, na1_ref):
    hn1 = _dot(adj_ref[...], ho_ref[...]) + hn_ref[...]
    hn1_ref[...] = hn1
    t = jax.nn.relu(_dot(hn1, p1_ref[...].T) + pb1_ref[...])
    na1_ref[...] = _dot(t, p2_ref[...].T) + pb2_ref[...]


def _full(shape):
    return pl.BlockSpec(shape, lambda i: tuple(0 for _ in shape))


def _rows(bs, width):
    return pl.BlockSpec((bs, width), lambda i: (i, 0))


def kernel(x, x_net, net_inst_adj, inst_net_adj_v_drive, inst_net_adj_v_sink,
           batch, num_vn, params):
    p = params
    r2 = lambda a: a.reshape(1, -1)
    oh = (batch[:, None] == jnp.arange(NUM_VN, dtype=batch.dtype)[None, :]
          ).astype(jnp.float32)
    vn0 = jnp.tile(p["vn_emb"], (NUM_VN, 1)) + 0.0 * num_vn

    h0 = pl.pallas_call(
        _enc_inst_body,
        grid=(8,),
        in_specs=[_rows(N_INST // 8, x.shape[1]),
                  _full(p["enc_W1"].shape), _full((1, 2 * EMB)),
                  _full(p["enc_W2"].shape), _full((1, EMB))],
        out_specs=_rows(N_INST // 8, EMB),
        out_shape=jax.ShapeDtypeStruct((N_INST, EMB), jnp.float32),
    )(x, p["enc_W1"], r2(p["enc_b1"]), p["enc_W2"], r2(p["enc_b2"]))

    L0, L1 = p["layers"][0], p["layers"][1]
    hn0, na0 = pl.pallas_call(
        _enc_net_body,
        grid=(4,),
        in_specs=[_rows(N_NET // 4, x_net.shape[1]),
                  _full(p["encnet_W1"].shape), _full((1, EMB)),
                  _full(p["encnet_W2"].shape), _full((1, EMB)),
                  _full(L0["phi_W1"].shape), _full((1, EMB)),
                  _full(L0["phi_W2"].shape), _full((1, EMB))],
        out_specs=[_rows(N_NET // 4, EMB), _rows(N_NET // 4, EMB)],
        out_shape=[jax.ShapeDtypeStruct((N_NET, EMB), jnp.float32),
                   jax.ShapeDtypeStruct((N_NET, EMB), jnp.float32)],
    )(x_net, p["encnet_W1"], r2(p["encnet_b1"]), p["encnet_W2"],
      r2(p["encnet_b2"]), L0["phi_W1"], r2(p["layers"][0]["phi_b1"]),
      L0["phi_W2"], r2(L0["phi_b2"]))

    def layer_call(L, q, drive, sink, na, h_base, vn_table, do_vn):
        grid = (N_INST // BR,)
        return pl.pallas_call(
            functools.partial(_layer_body, do_vn=do_vn),
            grid=grid,
            in_specs=[_rows(BR, N_NET), _rows(BR, N_NET), _full((N_NET, EMB)),
                      _rows(BR, EMB), _rows(BR, NUM_VN), _full((NUM_VN, EMB)),
                      _full(L["psi_W1"].shape), _full((1, EMB)),
                      _full(L["psi_W2"].shape), _full((1, EMB)),
                      _full(L["mlp_W1"].shape), _full((1, 3 * EMB)),
                      _full(L["mlp_W2"].shape), _full((1, EMB)),
                      _full((1, EMB)), _full((1, EMB)),
                      _full(q["W1"].shape), _full((1, 2 * EMB)),
                      _full(q["W2"].shape), _full((1, EMB))],
            out_specs=[_rows(BR, EMB), _rows(BR, EMB), _rows(BR, EMB),
                       _full((NUM_VN, 2 * EMB)), _full((NUM_VN, EMB))],
            out_shape=[jax.ShapeDtypeStruct((N_INST, EMB), jnp.float32),
                       jax.ShapeDtypeStruct((N_INST, EMB), jnp.float32),
                       jax.ShapeDtypeStruct((N_INST, EMB), jnp.float32),
                       jax.ShapeDtypeStruct((NUM_VN, 2 * EMB), jnp.float32),
                       jax.ShapeDtypeStruct((NUM_VN, EMB), jnp.float32)],
        )(drive, sink, na, h_base, oh, vn_table,
          L["psi_W1"], r2(L["psi_b1"]), L["psi_W2"], r2(L["psi_b2"]),
          L["mlp_W1"], r2(L["mlp_b1"]), L["mlp_W2"], r2(L["mlp_b2"]),
          r2(L["ln_g"]), r2(L["ln_b"]),
          q["W1"], r2(q["b1"]), q["W2"], r2(q["b2"]))

    q0 = p["vn_mlp"][0]
    h_out0, h_in0, h_pre0, _, vn1 = layer_call(
        L0, q0, inst_net_adj_v_drive, inst_net_adj_v_sink, na0, h0, vn0, True)

    hn1, na1 = pl.pallas_call(
        _net_body,
        grid=(N_NET // BN,),
        in_specs=[_rows(BN, N_INST), _full((N_INST, EMB)), _rows(BN, EMB),
                  _full(L1["phi_W1"].shape), _full((1, EMB)),
                  _full(L1["phi_W2"].shape), _full((1, EMB))],
        out_specs=[_rows(BN, EMB), _rows(BN, EMB)],
        out_shape=[jax.ShapeDtypeStruct((N_NET, EMB), jnp.float32),
                   jax.ShapeDtypeStruct((N_NET, EMB), jnp.float32)],
    )(net_inst_adj, h_pre0, hn0, L1["phi_W1"], r2(L1["phi_b1"]),
      L1["phi_W2"], r2(L1["phi_b2"]))

    h_out1, h_in1, _, _, _ = layer_call(
        L1, q0, inst_net_adj_v_drive, inst_net_adj_v_sink, na1, h_out0, vn1,
        False)

    return jnp.concatenate([h_in0, h_in1, h_out1], axis=1)
